# baseline (speedup 1.0000x reference)
import functools

import jax
import jax.numpy as jnp
import numpy as np
from jax import lax
from jax.experimental import pallas as pl
from jax.experimental.pallas import tpu as pltpu
from jax.experimental.pallas import tpu_sc as plsc

N_NODES = 10000
N_EDGES = 320000
D_FEAT = 128
D_HID = 64
N_CLASSES = 16
EPOCHS = 5
LR = 0.01
EPS = 1e-15
C_ES = 0.005
C_NF = 1.0
C_EE = 1.0
C_NE = 0.1

NPAD = 10240
NW = 16
E2 = 327680
EROWS = E2 // 128
NB = 10
EA = 4096
NH = 128
NSLICE = NPAD // NW

_mesh = plsc.VectorSubcoreMesh(core_axis_name="c", subcore_axis_name="s",
                               num_cores=1)


@functools.partial(
    pl.kernel, mesh=_mesh,
    out_type=(
        jax.ShapeDtypeStruct((NPAD,), jnp.float32),
        jax.ShapeDtypeStruct((NPAD,), jnp.float32),
        jax.ShapeDtypeStruct((NPAD,), jnp.float32),
        jax.ShapeDtypeStruct((NPAD,), jnp.float32),
        jax.ShapeDtypeStruct((E2,), jnp.float32),
        jax.ShapeDtypeStruct((E2,), jnp.float32),
    ),
    scratch_types=[
        pltpu.VMEM((16, 128), jnp.int32),
        pltpu.VMEM((16, 128), jnp.int32),
        pltpu.VMEM((2048,), jnp.float32),
        pltpu.VMEM((128,), jnp.float32),
        pltpu.VMEM((2048,), jnp.float32),
        pltpu.VMEM((2048,), jnp.float32),
        pltpu.VMEM((2048,), jnp.float32),
        pltpu.VMEM((2048,), jnp.float32),
        pltpu.VMEM((2048,), jnp.float32),
        pltpu.VMEM((NSLICE,), jnp.float32),
        pltpu.VMEM((NSLICE,), jnp.float32),
        pltpu.VMEM((16,), jnp.int32),
        pltpu.VMEM_SHARED((NPAD,), jnp.float32),
        pltpu.VMEM_SHARED((NPAD,), jnp.float32),
        pltpu.VMEM_SHARED((NPAD,), jnp.float32),
        pltpu.VMEM_SHARED((NPAD,), jnp.float32),
        pltpu.VMEM_SHARED((NPAD,), jnp.float32),
        pltpu.VMEM_SHARED((NPAD,), jnp.float32),
        pltpu.SemaphoreType.DMA,
    ],
    compiler_params=pltpu.CompilerParams(needs_layout_passes=False),
)
def _sc_graph(src_hbm, dst_hbm, ni_hbm,
              degF_hbm, r1_hbm, nm_hbm, ds_hbm, em_hbm, act_hbm,
              sidx, didx, vals, ones128, gbufA, gbufB, gbufC, embuf, actbuf,
              nodebuf, nodebuf2, nib, acc_deg, acc_r1, acc_fr, acc_r2,
              acc_nm, acc_ds, sem):
    wid = lax.axis_index("s")
    zero16 = jnp.zeros((16,), jnp.float32)
    one16 = jnp.ones((16,), jnp.float32)

    pltpu.sync_copy(ni_hbm, nib)
    ni = nib[pl.ds(0, 16)][0]

    def fill16(i, _):
        nodebuf[pl.ds(i * 16, 16)] = zero16
        return 0
    lax.fori_loop(0, NSLICE // 16, fill16, 0)

    def fillones(i, _):
        ones128[pl.ds(i * 16, 16)] = one16
        return 0
    lax.fori_loop(0, 8, fillones, 0)

    nslc = pl.ds(wid * NSLICE, NSLICE)
    for acc in (acc_deg, acc_r1, acc_fr, acc_r2, acc_nm, acc_ds):
        pltpu.sync_copy(nodebuf, acc.at[nslc])
    plsc.subcore_barrier()

    def p1(b, _):
        row0 = wid * (NB * 16) + b * 16
        pltpu.sync_copy(src_hbm.at[pl.ds(row0, 16)], sidx)
        pltpu.sync_copy(dst_hbm.at[pl.ds(row0, 16)], didx)

        def cmp(i, _):
            r = i // 8
            k = i % 8
            dv = didx[r, pl.ds(k * 16, 16)]
            vals[pl.ds(i * 16, 16)] = jnp.where(dv == ni, 1.0, 0.0)
            return 0
        lax.fori_loop(0, 128, cmp, 0)
        hs = []
        for j in range(16):
            hs.append(pltpu.async_copy(ones128, acc_deg.at[didx.at[j]],
                                       sem, add=True))
            hs.append(pltpu.async_copy(vals.at[pl.ds(j * 128, 128)],
                                       acc_r1.at[sidx.at[j]], sem, add=True))
        for h in hs:
            h.wait()
        return 0
    lax.fori_loop(0, NB, p1, 0)
    plsc.subcore_barrier()

    pltpu.sync_copy(acc_r1.at[nslc], nodebuf)

    def fr(i, _):
        v = nodebuf[pl.ds(i * 16, 16)]
        idxv = lax.iota(jnp.int32, 16) + (wid * NSLICE + i * 16)
        nodebuf[pl.ds(i * 16, 16)] = jnp.where((v > 0.0) & (idxv != ni),
                                               1.0, 0.0)
        return 0
    lax.fori_loop(0, NSLICE // 16, fr, 0)
    pltpu.sync_copy(nodebuf, acc_fr.at[nslc])
    plsc.subcore_barrier()

    def p2(b, _):
        row0 = wid * (NB * 16) + b * 16
        pltpu.sync_copy(src_hbm.at[pl.ds(row0, 16)], sidx)
        pltpu.sync_copy(dst_hbm.at[pl.ds(row0, 16)], didx)
        hs = []
        for j in range(16):
            hs.append(pltpu.async_copy(
                acc_fr.at[didx.at[j]], gbufB.at[pl.ds(j * 128, 128)], sem))
        for h in hs:
            h.wait()
        hs2 = []
        for j in range(16):
            hs2.append(pltpu.async_copy(gbufB.at[pl.ds(j * 128, 128)],
                                        acc_r2.at[sidx.at[j]], sem, add=True))
        for h in hs2:
            h.wait()
        return 0
    lax.fori_loop(0, NB, p2, 0)
    plsc.subcore_barrier()

    pltpu.sync_copy(acc_r1.at[nslc], nodebuf)
    pltpu.sync_copy(acc_r2.at[nslc], nodebuf2)

    def nm(i, _):
        v1 = nodebuf[pl.ds(i * 16, 16)]
        v2 = nodebuf2[pl.ds(i * 16, 16)]
        idxv = lax.iota(jnp.int32, 16) + (wid * NSLICE + i * 16)
        nodebuf[pl.ds(i * 16, 16)] = jnp.where(
            (v1 > 0.0) | (v2 > 0.0) | (idxv == ni), 1.0, 0.0)
        return 0
    lax.fori_loop(0, NSLICE // 16, nm, 0)
    pltpu.sync_copy(nodebuf, acc_nm.at[nslc])
    plsc.subcore_barrier()

    def p3(b, _):
        row0 = wid * (NB * 16) + b * 16
        pltpu.sync_copy(src_hbm.at[pl.ds(row0, 16)], sidx)
        pltpu.sync_copy(dst_hbm.at[pl.ds(row0, 16)], didx)
        hs = []
        for j in range(16):
            hs.append(pltpu.async_copy(
                acc_nm.at[sidx.at[j]], gbufA.at[pl.ds(j * 128, 128)], sem))
            hs.append(pltpu.async_copy(
                acc_nm.at[didx.at[j]], gbufB.at[pl.ds(j * 128, 128)], sem))
            hs.append(pltpu.async_copy(
                acc_r1.at[didx.at[j]], gbufC.at[pl.ds(j * 128, 128)], sem))
        for h in hs:
            h.wait()

        def cmp(i, _):
            r = i // 8
            k = i % 8
            a = gbufA[pl.ds(i * 16, 16)]
            bb = gbufB[pl.ds(i * 16, 16)]
            c = gbufC[pl.ds(i * 16, 16)]
            dv = didx[r, pl.ds(k * 16, 16)]
            em = a * bb
            embuf[pl.ds(i * 16, 16)] = em
            actbuf[pl.ds(i * 16, 16)] = em * jnp.where(
                (c > 0.0) | (dv == ni), 1.0, 0.0)
            return 0
        lax.fori_loop(0, 128, cmp, 0)
        hs2 = []
        for j in range(16):
            hs2.append(pltpu.async_copy(embuf.at[pl.ds(j * 128, 128)],
                                        acc_ds.at[didx.at[j]], sem, add=True))
        for h in hs2:
            h.wait()
        ebase = (wid * (NB * 16) + b * 16) * 128
        pltpu.sync_copy(embuf, em_hbm.at[pl.ds(ebase, 2048)])
        pltpu.sync_copy(actbuf, act_hbm.at[pl.ds(ebase, 2048)])
        return 0
    lax.fori_loop(0, NB, p3, 0)
    plsc.subcore_barrier()

    pltpu.sync_copy(acc_deg.at[nslc], degF_hbm.at[nslc])
    pltpu.sync_copy(acc_r1.at[nslc], r1_hbm.at[nslc])
    pltpu.sync_copy(acc_nm.at[nslc], nm_hbm.at[nslc])
    pltpu.sync_copy(acc_ds.at[nslc], ds_hbm.at[nslc])


@functools.partial(
    pl.kernel, mesh=_mesh,
    out_type=(
        jax.ShapeDtypeStruct((EA, D_FEAT), jnp.float32),
        jax.ShapeDtypeStruct((E2,), jnp.float32),
    ),
    scratch_types=[
        pltpu.VMEM((1, 128), jnp.int32),
        pltpu.VMEM((128, D_FEAT), jnp.float32),
        pltpu.VMEM((16, 128), jnp.int32),
        pltpu.VMEM((2048,), jnp.float32),
        pltpu.SemaphoreType.DMA,
    ],
    compiler_params=pltpu.CompilerParams(needs_layout_passes=False),
)
def _sc_gather_rows(x_hbm, idx_hbm, rank_hbm, draw_hbm, out_hbm, gd_hbm,
                    idxv, rows, ridx, gbuf, sem):
    wid = lax.axis_index("s")
    for c in range(EA // 128 // NW):
        r = wid * (EA // 128 // NW) + c
        pltpu.sync_copy(idx_hbm.at[pl.ds(r, 1)], idxv)
        pltpu.async_copy(x_hbm.at[idxv.at[0]], rows, sem).wait()
        pltpu.sync_copy(rows, out_hbm.at[pl.ds(r * 128, 128)])

    def gb(b, _):
        row0 = wid * (NB * 16) + b * 16
        pltpu.sync_copy(rank_hbm.at[pl.ds(row0, 16)], ridx)
        hs = []
        for j in range(16):
            hs.append(pltpu.async_copy(
                draw_hbm.at[ridx.at[j]], gbuf.at[pl.ds(j * 128, 128)], sem))
        for h in hs:
            h.wait()
        pltpu.sync_copy(gbuf, gd_hbm.at[pl.ds(row0 * 128, 2048)])
        return 0
    lax.fori_loop(0, NB, gb, 0)


def _sigmoid(z):
    return 1.0 / (1.0 + jnp.exp(-z))


def _tc_train_body(cxT_ref, d1_ref, d2_ref, aL_ref, aF_ref, f1_ref, f2_ref,
                   vr_ref, emar_ref, gdraw_ref, maskf_ref, nmask_ref,
                   nfm0_ref, W1_ref, W2_ref,
                   nfm_out, fem_out, siga_out):
    cxT = cxT_ref[...]
    d1 = d1_ref[...]
    d2 = d2_ref[...]
    aL = aL_ref[...]
    aF = aF_ref[...]
    f1 = f1_ref[...]
    f2 = f2_ref[...]
    vr = vr_ref[...]
    emar = emar_ref[...]
    gdraw = gdraw_ref[...]
    maskf = maskf_ref[...]
    W1 = W1_ref[...]
    W2 = W2_ref[...]

    n_sub = jnp.sum(nmask_ref[...])
    std_edge = jnp.sqrt(2.0 / n_sub)
    e_cnt = jnp.sum(maskf)

    iota_n = lax.broadcasted_iota(jnp.int32, (NH, EA), 0)
    c1 = jnp.where(iota_n == d1, 1.0, 0.0) * (f1 * vr)
    c2 = jnp.where(iota_n == d2, 1.0, 0.0) * (f2 * vr)
    S1L = c1 * aL
    S1F = c1 * aF
    S2L = c2 * aL
    S2F = c2 * aF

    def fwd(S1, S2, w, s):
        Cw = cxT * w
        P = lax.dot_general(S1, Cw, (((1,), (1,)), ((), ())))
        z1 = jnp.dot(P * s, W1)
        h1 = jnp.maximum(z1, 0.0)
        g2 = jnp.dot(h1, W2)
        q2 = lax.dot_general(S2, w, (((1,), (1,)), ((), ())))
        out = lax.dot_general(q2, g2, (((0,), (0,)), ((), ())))
        return P, z1, g2, q2, out

    ones_e = jnp.ones((1, EA), jnp.float32)
    ones_f = jnp.ones((1, D_FEAT), jnp.float32)
    _, _, _, _, out0 = fwd(S1F, S2F, ones_e, ones_f)
    m0 = jnp.max(out0)
    tgt = jnp.where(out0 == m0, 1.0, 0.0)
    tgt = tgt / jnp.sum(tgt)

    em_act = emar * std_edge
    em_full = maskf * (std_edge * gdraw)
    nfm = nfm0_ref[...]

    b1, b2, aeps = 0.9, 0.999, 1e-8
    mA_a = jnp.zeros_like(em_act)
    vA_a = jnp.zeros_like(em_act)
    mA_f = jnp.zeros_like(em_full)
    vA_f = jnp.zeros_like(em_full)
    mA_n = jnp.zeros_like(nfm)
    vA_n = jnp.zeros_like(nfm)

    for t in range(1, EPOCHS + 1):
        s = _sigmoid(nfm)
        w = _sigmoid(em_act)
        P, z1, g2, q2, out = fwd(S1L, S2L, w, s)
        mo = jnp.max(out)
        eo = jnp.exp(out - mo)
        p = eo / jnp.sum(eo)
        dout = p - tgt

        gd = lax.dot_general(g2, dout, (((1,), (1,)), ((), ())))
        dw2 = lax.dot_general(gd, S2L, (((0,), (0,)), ((), ())))
        dg2 = lax.dot_general(q2, dout, (((1,), (0,)), ((), ())))
        dh1 = lax.dot_general(dg2, W2, (((1,), (1,)), ((), ())))
        dz1 = dh1 * jnp.where(z1 > 0.0, 1.0, 0.0)
        rv = lax.dot_general(dz1, W1, (((1,), (1,)), ((), ())))
        ds_main = jnp.sum(P * rv, axis=0, keepdims=True)
        dP = rv * s
        U = lax.dot_general(dP, S1L, (((0,), (0,)), ((), ())))
        dw1 = jnp.sum(U * cxT, axis=0, keepdims=True)

        mAs = _sigmoid(em_act)
        dentA = jnp.log(1.0 - mAs + EPS) - jnp.log(mAs + EPS)
        gA = ((dw1 + dw2) + vr * (C_ES + C_EE * dentA / e_cnt)) \
            * mAs * (1.0 - mAs)

        mFs = _sigmoid(em_full)
        dentF = jnp.log(1.0 - mFs + EPS) - jnp.log(mFs + EPS)
        gF = maskf * (C_ES + C_EE * dentF / e_cnt) * mFs * (1.0 - mFs)

        dentN = jnp.log(1.0 - s + EPS) - jnp.log(s + EPS)
        gN = (ds_main + C_NF + C_NE * dentN / D_FEAT) * s * (1.0 - s)

        c1t = 1.0 - b1 ** t
        c2t = 1.0 - b2 ** t

        mA_a = b1 * mA_a + (1 - b1) * gA
        vA_a = b2 * vA_a + (1 - b2) * gA * gA
        em_act = em_act - LR * (mA_a / c1t) / (jnp.sqrt(vA_a / c2t) + aeps)

        mA_f = b1 * mA_f + (1 - b1) * gF
        vA_f = b2 * vA_f + (1 - b2) * gF * gF
        em_full = em_full - LR * (mA_f / c1t) / (jnp.sqrt(vA_f / c2t) + aeps)

        mA_n = b1 * mA_n + (1 - b1) * gN
        vA_n = b2 * vA_n + (1 - b2) * gN * gN
        nfm = nfm - LR * (mA_n / c1t) / (jnp.sqrt(vA_n / c2t) + aeps)

    nfm_out[...] = _sigmoid(nfm)
    fem_out[...] = maskf * _sigmoid(em_full)
    siga_out[...] = _sigmoid(em_act)


def _tc_train(cxT, d1, d2, aL, aF, f1, f2, vr, emar, gdraw2, maskf2, nmaskp,
              nfm0, W1, W2):
    return pl.pallas_call(
        _tc_train_body,
        out_shape=(
            jax.ShapeDtypeStruct((1, D_FEAT), jnp.float32),
            jax.ShapeDtypeStruct((N_EDGES // 128, 128), jnp.float32),
            jax.ShapeDtypeStruct((1, EA), jnp.float32),
        ),
    )(cxT, d1, d2, aL, aF, f1, f2, vr, emar, gdraw2, maskf2, nmaskp,
      nfm0, W1, W2)


def kernel(x, edge_index, node_idx, W1, W2):
    src = edge_index[0]
    dst = edge_index[1]
    ni = jnp.asarray(node_idx, jnp.int32)

    pad = jnp.full((E2 - N_EDGES,), N_NODES, jnp.int32)
    src2d = jnp.concatenate([src, pad]).reshape(EROWS, 128)
    dst2d = jnp.concatenate([dst, pad]).reshape(EROWS, 128)
    ni16 = jnp.full((16,), ni, jnp.int32)

    degF, r1, nmf, dsub, emE2, actE2 = _sc_graph(src2d, dst2d, ni16)
    emask = emE2[:N_EDGES]
    act_flag = actE2[:N_EDGES]

    h1rel = jnp.cumsum((r1 > 0).astype(jnp.int32)) - 1
    rank = jnp.cumsum(emask.astype(jnp.int32)) - 1

    mk1, mk2 = jax.random.split(jax.random.key(1))
    nfm0 = 0.1 * jax.random.normal(mk1, (D_FEAT,), dtype=jnp.float32)
    draw = jax.random.normal(mk2, (N_EDGES,), dtype=jnp.float32)

    ap_raw = jnp.nonzero(act_flag > 0.0, size=EA, fill_value=-1)[0]
    valid = ap_raw >= 0
    ap = jnp.where(valid, ap_raw, 0)
    a_src = src[ap]
    a_dst = dst[ap]
    f1b = valid & (r1[a_dst] > 0)
    f2b = valid & (a_dst == ni)
    d1 = jnp.clip(jnp.where(f1b, h1rel[a_dst], 0), 0, NH - 1)
    d2 = jnp.clip(jnp.where(f2b, h1rel[a_src], 0), 0, NH - 1)
    dinvF = jnp.where(degF > 0, lax.rsqrt(jnp.maximum(degF, 1.0)), 0.0)
    dinvL = jnp.where(dsub > 0, lax.rsqrt(jnp.maximum(dsub, 1.0)), 0.0)
    vF = jnp.where(valid, dinvF[a_src] * dinvF[a_dst], 0.0)
    vL = jnp.where(valid, dinvL[a_src] * dinvL[a_dst], 0.0)
    aidx2d = jnp.where(valid, a_src, 0).astype(jnp.int32).reshape(EA // 128,
                                                                  128)
    rankc = jnp.clip(rank, 0, N_EDGES - 1)
    rank2d = jnp.concatenate(
        [rankc, jnp.zeros((E2 - N_EDGES,), jnp.int32)]).reshape(EROWS, 128)
    cx, gdE2 = _sc_gather_rows(x, aidx2d, rank2d, draw)
    gdraw = gdE2[:N_EDGES]
    emar = jnp.where(valid, gdraw[ap], 0.0)
    cxT = cx.T

    row = lambda a, dt: a.astype(dt).reshape(1, EA)
    nfm_sig, fem, sig_act = _tc_train(
        cxT, row(d1, jnp.int32), row(d2, jnp.int32), row(vL, jnp.float32),
        row(vF, jnp.float32), row(f1b, jnp.float32), row(f2b, jnp.float32),
        row(valid, jnp.float32), row(emar, jnp.float32),
        gdraw.reshape(N_EDGES // 128, 128),
        emask.reshape(N_EDGES // 128, 128),
        nmf.reshape(NPAD // 128, 128),
        nfm0.reshape(1, D_FEAT), W1, W2)

    full = fem.reshape(N_EDGES)
    scat = jnp.where(valid, ap_raw, N_EDGES)
    full = full.at[scat].set(sig_act.reshape(EA), mode="drop")
    return nfm_sig.reshape(D_FEAT), full


# speedup vs baseline: 1.1571x; 1.1571x over previous
import functools

import jax
import jax.numpy as jnp
import numpy as np
from jax import lax
from jax.experimental import pallas as pl
from jax.experimental.pallas import tpu as pltpu
from jax.experimental.pallas import tpu_sc as plsc

N_NODES = 10000
N_EDGES = 320000
D_FEAT = 128
D_HID = 64
N_CLASSES = 16
EPOCHS = 5
LR = 0.01
EPS = 1e-15
C_ES = 0.005
C_NF = 1.0
C_EE = 1.0
C_NE = 0.1

NPAD = 10240
NW = 16
E2 = 327680
EROWS = E2 // 128
NB = 10
EA = 4096
NH = 128
NSLICE = NPAD // NW

_mesh = plsc.VectorSubcoreMesh(core_axis_name="c", subcore_axis_name="s",
                               num_cores=1)


@functools.partial(
    pl.kernel, mesh=_mesh,
    out_type=(
        jax.ShapeDtypeStruct((NPAD,), jnp.float32),
        jax.ShapeDtypeStruct((NPAD,), jnp.float32),
        jax.ShapeDtypeStruct((NPAD,), jnp.float32),
        jax.ShapeDtypeStruct((NPAD,), jnp.float32),
        jax.ShapeDtypeStruct((E2,), jnp.float32),
        jax.ShapeDtypeStruct((E2,), jnp.float32),
    ),
    scratch_types=[
        pltpu.VMEM((16, 128), jnp.int32),
        pltpu.VMEM((16, 128), jnp.int32),
        pltpu.VMEM((2048,), jnp.float32),
        pltpu.VMEM((128,), jnp.float32),
        pltpu.VMEM((2048,), jnp.float32),
        pltpu.VMEM((2048,), jnp.float32),
        pltpu.VMEM((2048,), jnp.float32),
        pltpu.VMEM((2048,), jnp.float32),
        pltpu.VMEM((2048,), jnp.float32),
        pltpu.VMEM((NSLICE,), jnp.float32),
        pltpu.VMEM((NSLICE,), jnp.float32),
        pltpu.VMEM((16,), jnp.int32),
        pltpu.VMEM_SHARED((NPAD,), jnp.float32),
        pltpu.VMEM_SHARED((NPAD,), jnp.float32),
        pltpu.VMEM_SHARED((NPAD,), jnp.float32),
        pltpu.VMEM_SHARED((NPAD,), jnp.float32),
        pltpu.VMEM_SHARED((NPAD,), jnp.float32),
        pltpu.VMEM_SHARED((NPAD,), jnp.float32),
        pltpu.SemaphoreType.DMA,
    ],
    compiler_params=pltpu.CompilerParams(needs_layout_passes=False),
)
def _sc_graph(src_hbm, dst_hbm, ni_hbm,
              degF_hbm, r1_hbm, nm_hbm, ds_hbm, em_hbm, act_hbm,
              sidx, didx, vals, ones128, gbufA, gbufB, gbufC, embuf, actbuf,
              nodebuf, nodebuf2, nib, acc_deg, acc_r1, acc_fr, acc_r2,
              acc_nm, acc_ds, sem):
    wid = lax.axis_index("s")
    zero16 = jnp.zeros((16,), jnp.float32)
    one16 = jnp.ones((16,), jnp.float32)

    pltpu.sync_copy(ni_hbm, nib)
    ni = nib[pl.ds(0, 16)][0]

    def fill16(i, _):
        nodebuf[pl.ds(i * 16, 16)] = zero16
        return 0
    lax.fori_loop(0, NSLICE // 16, fill16, 0)

    def fillones(i, _):
        ones128[pl.ds(i * 16, 16)] = one16
        return 0
    lax.fori_loop(0, 8, fillones, 0)

    nslc = pl.ds(wid * NSLICE, NSLICE)
    for acc in (acc_deg, acc_r1, acc_fr, acc_r2, acc_nm, acc_ds):
        pltpu.sync_copy(nodebuf, acc.at[nslc])
    plsc.subcore_barrier()

    def p1(b, _):
        row0 = wid * (NB * 16) + b * 16
        pltpu.sync_copy(src_hbm.at[pl.ds(row0, 16)], sidx)
        pltpu.sync_copy(dst_hbm.at[pl.ds(row0, 16)], didx)

        def cmp(i, _):
            r = i // 8
            k = i % 8
            dv = didx[r, pl.ds(k * 16, 16)]
            vals[pl.ds(i * 16, 16)] = jnp.where(dv == ni, 1.0, 0.0)
            return 0
        lax.fori_loop(0, 128, cmp, 0)
        hs = []
        for j in range(16):
            hs.append(pltpu.async_copy(ones128, acc_deg.at[didx.at[j]],
                                       sem, add=True))
            hs.append(pltpu.async_copy(vals.at[pl.ds(j * 128, 128)],
                                       acc_r1.at[sidx.at[j]], sem, add=True))
        for h in hs:
            h.wait()
        return 0
    lax.fori_loop(0, NB, p1, 0)
    plsc.subcore_barrier()

    pltpu.sync_copy(acc_r1.at[nslc], nodebuf)

    def fr(i, _):
        v = nodebuf[pl.ds(i * 16, 16)]
        idxv = lax.iota(jnp.int32, 16) + (wid * NSLICE + i * 16)
        nodebuf[pl.ds(i * 16, 16)] = jnp.where((v > 0.0) & (idxv != ni),
                                               1.0, 0.0)
        return 0
    lax.fori_loop(0, NSLICE // 16, fr, 0)
    pltpu.sync_copy(nodebuf, acc_fr.at[nslc])
    plsc.subcore_barrier()

    def p2(b, _):
        row0 = wid * (NB * 16) + b * 16
        pltpu.sync_copy(src_hbm.at[pl.ds(row0, 16)], sidx)
        pltpu.sync_copy(dst_hbm.at[pl.ds(row0, 16)], didx)
        hs = []
        for j in range(16):
            hs.append(pltpu.async_copy(
                acc_fr.at[didx.at[j]], gbufB.at[pl.ds(j * 128, 128)], sem))
        for h in hs:
            h.wait()
        hs2 = []
        for j in range(16):
            hs2.append(pltpu.async_copy(gbufB.at[pl.ds(j * 128, 128)],
                                        acc_r2.at[sidx.at[j]], sem, add=True))
        for h in hs2:
            h.wait()
        return 0
    lax.fori_loop(0, NB, p2, 0)
    plsc.subcore_barrier()

    pltpu.sync_copy(acc_r1.at[nslc], nodebuf)
    pltpu.sync_copy(acc_r2.at[nslc], nodebuf2)

    def nm(i, _):
        v1 = nodebuf[pl.ds(i * 16, 16)]
        v2 = nodebuf2[pl.ds(i * 16, 16)]
        idxv = lax.iota(jnp.int32, 16) + (wid * NSLICE + i * 16)
        nodebuf[pl.ds(i * 16, 16)] = jnp.where(
            (v1 > 0.0) | (v2 > 0.0) | (idxv == ni), 1.0, 0.0)
        return 0
    lax.fori_loop(0, NSLICE // 16, nm, 0)
    pltpu.sync_copy(nodebuf, acc_nm.at[nslc])
    plsc.subcore_barrier()

    def p3(b, _):
        row0 = wid * (NB * 16) + b * 16
        pltpu.sync_copy(src_hbm.at[pl.ds(row0, 16)], sidx)
        pltpu.sync_copy(dst_hbm.at[pl.ds(row0, 16)], didx)
        hs = []
        for j in range(16):
            hs.append(pltpu.async_copy(
                acc_nm.at[sidx.at[j]], gbufA.at[pl.ds(j * 128, 128)], sem))
            hs.append(pltpu.async_copy(
                acc_nm.at[didx.at[j]], gbufB.at[pl.ds(j * 128, 128)], sem))
            hs.append(pltpu.async_copy(
                acc_r1.at[didx.at[j]], gbufC.at[pl.ds(j * 128, 128)], sem))
        for h in hs:
            h.wait()

        def cmp(i, _):
            r = i // 8
            k = i % 8
            a = gbufA[pl.ds(i * 16, 16)]
            bb = gbufB[pl.ds(i * 16, 16)]
            c = gbufC[pl.ds(i * 16, 16)]
            dv = didx[r, pl.ds(k * 16, 16)]
            em = a * bb
            embuf[pl.ds(i * 16, 16)] = em
            actbuf[pl.ds(i * 16, 16)] = em * jnp.where(
                (c > 0.0) | (dv == ni), 1.0, 0.0)
            return 0
        lax.fori_loop(0, 128, cmp, 0)
        hs2 = []
        for j in range(16):
            hs2.append(pltpu.async_copy(embuf.at[pl.ds(j * 128, 128)],
                                        acc_ds.at[didx.at[j]], sem, add=True))
        for h in hs2:
            h.wait()
        ebase = (wid * (NB * 16) + b * 16) * 128
        pltpu.sync_copy(embuf, em_hbm.at[pl.ds(ebase, 2048)])
        pltpu.sync_copy(actbuf, act_hbm.at[pl.ds(ebase, 2048)])
        return 0
    lax.fori_loop(0, NB, p3, 0)
    plsc.subcore_barrier()

    pltpu.sync_copy(acc_deg.at[nslc], degF_hbm.at[nslc])
    pltpu.sync_copy(acc_r1.at[nslc], r1_hbm.at[nslc])
    pltpu.sync_copy(acc_nm.at[nslc], nm_hbm.at[nslc])
    pltpu.sync_copy(acc_ds.at[nslc], ds_hbm.at[nslc])


@functools.partial(
    pl.kernel, mesh=_mesh,
    out_type=(
        jax.ShapeDtypeStruct((EA, D_FEAT), jnp.float32),
        jax.ShapeDtypeStruct((E2,), jnp.float32),
    ),
    scratch_types=[
        pltpu.VMEM((1, 128), jnp.int32),
        pltpu.VMEM((128, D_FEAT), jnp.float32),
        pltpu.VMEM((16, 128), jnp.int32),
        pltpu.VMEM((2048,), jnp.float32),
        pltpu.VMEM((N_EDGES // NW,), jnp.float32),
        pltpu.VMEM_SHARED((N_EDGES,), jnp.float32),
        pltpu.SemaphoreType.DMA,
    ],
    compiler_params=pltpu.CompilerParams(needs_layout_passes=False),
)
def _sc_gather_rows(x_hbm, idx_hbm, rank_hbm, draw_hbm, out_hbm, gd_hbm,
                    idxv, rows, ridx, gbuf, dbuf, draw_sh, sem):
    wid = lax.axis_index("s")
    dslc = pl.ds(wid * (N_EDGES // NW), N_EDGES // NW)
    pltpu.sync_copy(draw_hbm.at[dslc], dbuf)
    pltpu.sync_copy(dbuf, draw_sh.at[dslc])
    for c in range(EA // 128 // NW):
        r = wid * (EA // 128 // NW) + c
        pltpu.sync_copy(idx_hbm.at[pl.ds(r, 1)], idxv)
        pltpu.async_copy(x_hbm.at[idxv.at[0]], rows, sem).wait()
        pltpu.sync_copy(rows, out_hbm.at[pl.ds(r * 128, 128)])
    plsc.subcore_barrier()

    def gb(b, _):
        row0 = wid * (NB * 16) + b * 16
        pltpu.sync_copy(rank_hbm.at[pl.ds(row0, 16)], ridx)
        hs = []
        for j in range(16):
            hs.append(pltpu.async_copy(
                draw_sh.at[ridx.at[j]], gbuf.at[pl.ds(j * 128, 128)], sem))
        for h in hs:
            h.wait()
        pltpu.sync_copy(gbuf, gd_hbm.at[pl.ds(row0 * 128, 2048)])
        return 0
    lax.fori_loop(0, NB, gb, 0)


def _sigmoid(z):
    return 1.0 / (1.0 + jnp.exp(-z))


def _tc_train_body(cxT_ref, d1_ref, d2_ref, aL_ref, aF_ref, f1_ref, f2_ref,
                   vr_ref, emar_ref, gdraw_ref, maskf_ref, nmask_ref,
                   nfm0_ref, W1_ref, W2_ref,
                   nfm_out, fem_out, siga_out):
    cxT = cxT_ref[...]
    d1 = d1_ref[...]
    d2 = d2_ref[...]
    aL = aL_ref[...]
    aF = aF_ref[...]
    f1 = f1_ref[...]
    f2 = f2_ref[...]
    vr = vr_ref[...]
    emar = emar_ref[...]
    gdraw = gdraw_ref[...]
    maskf = maskf_ref[...]
    W1 = W1_ref[...]
    W2 = W2_ref[...]

    n_sub = jnp.sum(nmask_ref[...])
    std_edge = jnp.sqrt(2.0 / n_sub)
    e_cnt = jnp.sum(maskf)

    iota_n = lax.broadcasted_iota(jnp.int32, (NH, EA), 0)
    c1 = jnp.where(iota_n == d1, 1.0, 0.0) * (f1 * vr)
    c2 = jnp.where(iota_n == d2, 1.0, 0.0) * (f2 * vr)
    S1L = c1 * aL
    S1F = c1 * aF
    S2L = c2 * aL
    S2F = c2 * aF

    def fwd(S1, S2, w, s):
        Cw = cxT * w
        P = lax.dot_general(S1, Cw, (((1,), (1,)), ((), ())))
        z1 = jnp.dot(P * s, W1)
        h1 = jnp.maximum(z1, 0.0)
        g2 = jnp.dot(h1, W2)
        q2 = lax.dot_general(S2, w, (((1,), (1,)), ((), ())))
        out = lax.dot_general(q2, g2, (((0,), (0,)), ((), ())))
        return P, z1, g2, q2, out

    ones_e = jnp.ones((1, EA), jnp.float32)
    ones_f = jnp.ones((1, D_FEAT), jnp.float32)
    _, _, _, _, out0 = fwd(S1F, S2F, ones_e, ones_f)
    m0 = jnp.max(out0)
    tgt = jnp.where(out0 == m0, 1.0, 0.0)
    tgt = tgt / jnp.sum(tgt)

    em_act = emar * std_edge
    em_full = maskf * (std_edge * gdraw)
    nfm = nfm0_ref[...]

    b1, b2, aeps = 0.9, 0.999, 1e-8
    mA_a = jnp.zeros_like(em_act)
    vA_a = jnp.zeros_like(em_act)
    mA_f = jnp.zeros_like(em_full)
    vA_f = jnp.zeros_like(em_full)
    mA_n = jnp.zeros_like(nfm)
    vA_n = jnp.zeros_like(nfm)

    for t in range(1, EPOCHS + 1):
        s = _sigmoid(nfm)
        w = _sigmoid(em_act)
        P, z1, g2, q2, out = fwd(S1L, S2L, w, s)
        mo = jnp.max(out)
        eo = jnp.exp(out - mo)
        p = eo / jnp.sum(eo)
        dout = p - tgt

        gd = lax.dot_general(g2, dout, (((1,), (1,)), ((), ())))
        dw2 = lax.dot_general(gd, S2L, (((0,), (0,)), ((), ())))
        dg2 = lax.dot_general(q2, dout, (((1,), (0,)), ((), ())))
        dh1 = lax.dot_general(dg2, W2, (((1,), (1,)), ((), ())))
        dz1 = dh1 * jnp.where(z1 > 0.0, 1.0, 0.0)
        rv = lax.dot_general(dz1, W1, (((1,), (1,)), ((), ())))
        ds_main = jnp.sum(P * rv, axis=0, keepdims=True)
        dP = rv * s
        U = lax.dot_general(dP, S1L, (((0,), (0,)), ((), ())))
        dw1 = jnp.sum(U * cxT, axis=0, keepdims=True)

        mAs = _sigmoid(em_act)
        dentA = jnp.log(1.0 - mAs + EPS) - jnp.log(mAs + EPS)
        gA = ((dw1 + dw2) + vr * (C_ES + C_EE * dentA / e_cnt)) \
            * mAs * (1.0 - mAs)

        mFs = _sigmoid(em_full)
        dentF = jnp.log(1.0 - mFs + EPS) - jnp.log(mFs + EPS)
        gF = maskf * (C_ES + C_EE * dentF / e_cnt) * mFs * (1.0 - mFs)

        dentN = jnp.log(1.0 - s + EPS) - jnp.log(s + EPS)
        gN = (ds_main + C_NF + C_NE * dentN / D_FEAT) * s * (1.0 - s)

        c1t = 1.0 - b1 ** t
        c2t = 1.0 - b2 ** t

        mA_a = b1 * mA_a + (1 - b1) * gA
        vA_a = b2 * vA_a + (1 - b2) * gA * gA
        em_act = em_act - LR * (mA_a / c1t) / (jnp.sqrt(vA_a / c2t) + aeps)

        mA_f = b1 * mA_f + (1 - b1) * gF
        vA_f = b2 * vA_f + (1 - b2) * gF * gF
        em_full = em_full - LR * (mA_f / c1t) / (jnp.sqrt(vA_f / c2t) + aeps)

        mA_n = b1 * mA_n + (1 - b1) * gN
        vA_n = b2 * vA_n + (1 - b2) * gN * gN
        nfm = nfm - LR * (mA_n / c1t) / (jnp.sqrt(vA_n / c2t) + aeps)

    nfm_out[...] = _sigmoid(nfm)
    fem_out[...] = maskf * _sigmoid(em_full)
    siga_out[...] = _sigmoid(em_act)


def _tc_train(cxT, d1, d2, aL, aF, f1, f2, vr, emar, gdraw2, maskf2, nmaskp,
              nfm0, W1, W2):
    return pl.pallas_call(
        _tc_train_body,
        out_shape=(
            jax.ShapeDtypeStruct((1, D_FEAT), jnp.float32),
            jax.ShapeDtypeStruct((N_EDGES // 128, 128), jnp.float32),
            jax.ShapeDtypeStruct((1, EA), jnp.float32),
        ),
    )(cxT, d1, d2, aL, aF, f1, f2, vr, emar, gdraw2, maskf2, nmaskp,
      nfm0, W1, W2)


def kernel(x, edge_index, node_idx, W1, W2):
    src = edge_index[0]
    dst = edge_index[1]
    ni = jnp.asarray(node_idx, jnp.int32)

    pad = jnp.full((E2 - N_EDGES,), N_NODES, jnp.int32)
    src2d = jnp.concatenate([src, pad]).reshape(EROWS, 128)
    dst2d = jnp.concatenate([dst, pad]).reshape(EROWS, 128)
    ni16 = jnp.full((16,), ni, jnp.int32)

    degF, r1, nmf, dsub, emE2, actE2 = _sc_graph(src2d, dst2d, ni16)
    emask = emE2[:N_EDGES]
    act_flag = actE2[:N_EDGES]

    h1rel = jnp.cumsum((r1 > 0).astype(jnp.int32)) - 1
    rank = jnp.cumsum(emask.astype(jnp.int32)) - 1

    mk1, mk2 = jax.random.split(jax.random.key(1))
    nfm0 = 0.1 * jax.random.normal(mk1, (D_FEAT,), dtype=jnp.float32)
    draw = jax.random.normal(mk2, (N_EDGES,), dtype=jnp.float32)

    ap_raw = jnp.nonzero(act_flag > 0.0, size=EA, fill_value=-1)[0]
    valid = ap_raw >= 0
    ap = jnp.where(valid, ap_raw, 0)
    a_src = src[ap]
    a_dst = dst[ap]
    f1b = valid & (r1[a_dst] > 0)
    f2b = valid & (a_dst == ni)
    d1 = jnp.clip(jnp.where(f1b, h1rel[a_dst], 0), 0, NH - 1)
    d2 = jnp.clip(jnp.where(f2b, h1rel[a_src], 0), 0, NH - 1)
    dinvF = jnp.where(degF > 0, lax.rsqrt(jnp.maximum(degF, 1.0)), 0.0)
    dinvL = jnp.where(dsub > 0, lax.rsqrt(jnp.maximum(dsub, 1.0)), 0.0)
    vF = jnp.where(valid, dinvF[a_src] * dinvF[a_dst], 0.0)
    vL = jnp.where(valid, dinvL[a_src] * dinvL[a_dst], 0.0)
    aidx2d = jnp.where(valid, a_src, 0).astype(jnp.int32).reshape(EA // 128,
                                                                  128)
    rankc = jnp.clip(rank, 0, N_EDGES - 1)
    rank2d = jnp.concatenate(
        [rankc, jnp.zeros((E2 - N_EDGES,), jnp.int32)]).reshape(EROWS, 128)
    cx, gdE2 = _sc_gather_rows(x, aidx2d, rank2d, draw)
    gdraw = gdE2[:N_EDGES]
    emar = jnp.where(valid, gdraw[ap], 0.0)
    cxT = cx.T

    row = lambda a, dt: a.astype(dt).reshape(1, EA)
    nfm_sig, fem, sig_act = _tc_train(
        cxT, row(d1, jnp.int32), row(d2, jnp.int32), row(vL, jnp.float32),
        row(vF, jnp.float32), row(f1b, jnp.float32), row(f2b, jnp.float32),
        row(valid, jnp.float32), row(emar, jnp.float32),
        gdraw.reshape(N_EDGES // 128, 128),
        emask.reshape(N_EDGES // 128, 128),
        nmf.reshape(NPAD // 128, 128),
        nfm0.reshape(1, D_FEAT), W1, W2)

    full = fem.reshape(N_EDGES)
    scat = jnp.where(valid, ap_raw, N_EDGES)
    full = full.at[scat].set(sig_act.reshape(EA), mode="drop")
    return nfm_sig.reshape(D_FEAT), full
